# baseline (speedup 1.0000x reference)
import jax
import jax.numpy as jnp
from jax import lax
from jax.experimental import pallas as pl
from jax.experimental.pallas import tpu as pltpu
from jax.experimental.pallas import tpu_sc as plsc


def _exp_neg(v):
    q = jnp.maximum(v, -1.0) * jnp.float32(0.25)
    p = 1.0 + q * (1.0 / 6.0)
    p = 1.0 + q * (1.0 / 5.0) * p
    p = 1.0 + q * (1.0 / 4.0) * p
    p = 1.0 + q * (1.0 / 3.0) * p
    p = 1.0 + q * (1.0 / 2.0) * p
    p = 1.0 + q * p
    p = p * p
    return p * p


def _sc_mask_scale(wid, bidv, bbv, gxbuf, mbuf, out_ref, *, S, nsplit,
                   scale_base):
    half_rows = S // nsplit
    seg = half_rows * S
    b = (wid - scale_base) // nsplit
    half = (wid - scale_base) % nsplit
    y0 = half * half_rows

    for j in range(seg // 16):
        mbuf[pl.ds(j * 16, 16)] = jnp.zeros((16,), jnp.float32)

    lanes = jnp.arange(16, dtype=jnp.int32)
    sf = jnp.float32(S)

    def box(i, carry):
        bid_i = bidv[pl.ds(i, 16)][0]

        @pl.when(bid_i == b)
        def _():
            xc = (bbv[pl.ds(4 * i, 16)][0] * sf).astype(jnp.int32)
            yc = (bbv[pl.ds(4 * i + 1, 16)][0] * sf).astype(jnp.int32)
            wd = (bbv[pl.ds(4 * i + 2, 16)][0] * sf).astype(jnp.int32)
            ht = (bbv[pl.ds(4 * i + 3, 16)][0] * sf).astype(jnp.int32)
            xl = jnp.maximum(xc - wd // 2, 0)
            yt = jnp.maximum(yc - ht // 2, 0)
            xr = jnp.minimum(xc + wd // 2, S - 1)
            yd = jnp.minimum(yc + ht // 2, S - 1)
            w = (xr - xl + 1).astype(jnp.float32)
            h = (yd - yt + 1).astype(jnp.float32)
            xcg = xc.astype(jnp.float32)
            ycg = yc.astype(jnp.float32)
            wwv = jnp.full((16,), w, jnp.float32) * w
            hhv = jnp.full((16,), h, jnp.float32) * h
            for ci in range(S // 16):
                xs = lanes + (ci * 16)
                dxv = xs.astype(jnp.float32) - xcg
                gx = _exp_neg(-(dxv * dxv) / wwv)
                gx = jnp.where((xs >= xl) & (xs <= xr), gx, 0.0)
                gxbuf[pl.ds(ci * 16, 16)] = gx

            y_lo = jnp.maximum(yt, y0)
            y_hi = jnp.minimum(yd, y0 + half_rows - 1)

            def row(y, c2):
                dyf = y.astype(jnp.float32) - ycg
                dyv = jnp.full((16,), dyf, jnp.float32)
                gy = _exp_neg(-(dyv * dyv) / hhv)
                off = (y - y0) * S
                for ci in range(S // 16):
                    cur = mbuf[pl.ds(off + ci * 16, 16)]
                    gxc = gxbuf[pl.ds(ci * 16, 16)]
                    mbuf[pl.ds(off + ci * 16, 16)] = jnp.maximum(cur, gy * gxc)
                return c2

            lax.fori_loop(y_lo, y_hi + 1, row, 0)

        return carry

    lax.fori_loop(0, 64, box, 0)
    pltpu.sync_copy(mbuf.at[pl.ds(0, seg)],
                    out_ref.at[pl.ds(b * (nsplit * seg) + half * seg, seg)])


def _sc_mask_kernel(bid_ref, bb_ref, m0_ref, m1_ref, bidv, bbv, gxbuf, mbuf):
    wid = lax.axis_index("s") * 2 + lax.axis_index("c")
    pltpu.sync_copy(bid_ref, bidv)
    pltpu.sync_copy(bb_ref, bbv)

    _sc_mask_scale(wid, bidv, bbv, gxbuf, mbuf, m0_ref, S=80, nsplit=4,
                   scale_base=0)

    def scale1():
        _sc_mask_scale(wid, bidv, bbv, gxbuf, mbuf, m1_ref, S=40, nsplit=2,
                       scale_base=0)

    def noop():
        pass

    lax.cond(wid < 16, scale1, noop)


def _sc_masks(batch_idx, bboxes):
    bid = jnp.pad(batch_idx.astype(jnp.int32), (0, 16))
    bb = jnp.pad(bboxes.reshape(256), (0, 16))
    mesh = plsc.VectorSubcoreMesh(core_axis_name="c", subcore_axis_name="s")
    f = pl.kernel(
        _sc_mask_kernel,
        mesh=mesh,
        out_type=[
            jax.ShapeDtypeStruct((8 * 6400,), jnp.float32),
            jax.ShapeDtypeStruct((8 * 1600,), jnp.float32),
        ],
        scratch_types=[
            pltpu.VMEM((80,), jnp.int32),
            pltpu.VMEM((272,), jnp.float32),
            pltpu.VMEM((80,), jnp.float32),
            pltpu.VMEM((3200,), jnp.float32),
        ],
    )
    m0, m1 = f(bid, bb)
    return m0.reshape(8, 1, 6400), m1.reshape(8, 1, 1600)


def _dsum_kernel(p_ref, t_ref, d_ref):
    c = pl.program_id(1)
    d = p_ref[...] - t_ref[...]
    s = jnp.sum(d * d, axis=1, keepdims=True)

    @pl.when(c == 0)
    def _():
        d_ref[...] = s

    @pl.when(c != 0)
    def _():
        d_ref[...] += s


def _combine_kernel(m0_ref, d0_ref, m1_ref, d1_ref, o_ref):
    m0 = m0_ref[...]
    r0 = jnp.sum(m0 * m0 * d0_ref[...])
    sm0 = jnp.sum(m0)
    m1 = m1_ref[...]
    r1 = jnp.sum(m1 * m1 * d1_ref[...])
    sm1 = jnp.sum(m1)
    acc = r0 / (256.0 * sm0) + r1 / (256.0 * sm1)
    o_ref[0, 0] = 0.5 * acc


def _dsum(p, t, cb):
    B, C, S, _ = p.shape
    ss = S * S
    p = p.reshape(B, C, ss)
    t = t.reshape(B, C, ss)
    grid = (B, C // cb)
    return pl.pallas_call(
        _dsum_kernel,
        grid=grid,
        in_specs=[
            pl.BlockSpec((1, cb, ss), lambda b, c: (b, c, 0)),
            pl.BlockSpec((1, cb, ss), lambda b, c: (b, c, 0)),
        ],
        out_specs=pl.BlockSpec((1, 1, ss), lambda b, c: (b, 0, 0)),
        out_shape=jax.ShapeDtypeStruct((B, 1, ss), jnp.float32),
    )(p, t)


@jax.jit
def kernel(y_pred0, y_pred1, y_true0, y_true1, batch_idx, cls, bboxes):
    d0 = _dsum(y_pred0, y_true0, 256)
    d1 = _dsum(y_pred1, y_true1, 256)
    m0, m1 = _sc_masks(batch_idx, bboxes)
    out = pl.pallas_call(
        _combine_kernel,
        out_shape=jax.ShapeDtypeStruct((1, 1), jnp.float32),
        out_specs=pl.BlockSpec(memory_space=pltpu.SMEM),
    )(m0, d0, m1, d1)
    return out[0, 0]
